# baseline (speedup 1.0000x reference)
import functools

import jax
import jax.numpy as jnp
from jax import lax
from jax.experimental import pallas as pl
from jax.experimental.pallas import tpu as pltpu
from jax.experimental.pallas import tpu_sc as plsc

_SHAPE = (64, 128)
_N = _SHAPE[0] * _SHAPE[1]
_NC, _NS, _L = 2, 16, 16
_NW = _NC * _NS
_SEG = _N // _NW
_CHUNKS = _SEG // _L
_NBLK = _N // _L

_BETA = 0.9


def _any_nonzero(v):
    s = v[0]
    for j in range(1, _L):
        s = s + v[j]
    return s != 0.0


def _sc_body(sf_hbm, x_hbm, act_hbm, gain_hbm, thr_hbm, w2_hbm, out_hbm,
             sp_v, rows_v, acc_v, x_v, a_v, g_v, t_v, o_v, sem):
    wid = lax.axis_index("s") * _NC + lax.axis_index("c")
    base = wid * _SEG

    pltpu.sync_copy(sf_hbm, sp_v)
    pltpu.sync_copy(x_hbm.at[pl.ds(base, _SEG)], x_v)
    pltpu.sync_copy(act_hbm.at[pl.ds(base, _SEG)], a_v)
    pltpu.sync_copy(gain_hbm.at[pl.ds(base, _SEG)], g_v)
    pltpu.sync_copy(thr_hbm.at[pl.ds(base, _SEG)], t_v)

    def or_body(b, acc):
        sf = sp_v[pl.ds(b * _L, _L)]
        return jnp.maximum(acc, sf)

    anyv = lax.fori_loop(0, _NBLK, or_body, jnp.zeros((_L,), jnp.float32))
    has_spikes = _any_nonzero(anyv)

    zeros = jnp.zeros((_L,), jnp.float32)
    for k in range(_CHUNKS):
        acc_v[pl.ds(k * _L, _L)] = zeros

    @pl.when(has_spikes)
    def _heavy():
        def chunk_body(c, carry):
            blk = c * _L
            sf = sp_v[pl.ds(blk, _L)]
            active = _any_nonzero(sf)

            @pl.when(active)
            def _():
                iv = blk + lax.iota(jnp.int32, _L)
                pltpu.async_copy(w2_hbm.at[iv * _NW + wid], rows_v,
                                 sem).wait()
                for j in range(_L):
                    @pl.when(sf[j] != 0.0)
                    def _():
                        def add_chunk(k, cc):
                            sl = pl.ds(k * _L, _L)
                            acc_v[sl] = acc_v[sl] + rows_v[j, sl]
                            return cc
                        lax.fori_loop(0, _CHUNKS, add_chunk, 0)
            return carry

        lax.fori_loop(0, _NBLK, chunk_body, 0)

    for k in range(_CHUNKS):
        sl = pl.ds(k * _L, _L)
        gg = g_v[sl]
        ig = gg + (1.0 - gg) * 0.2
        act = _BETA * a_v[sl] + (x_v[sl] + acc_v[sl]) * ig + 0.05
        o_v[sl] = jnp.where(act > t_v[sl], 1.0, 0.0)
    pltpu.sync_copy(o_v, out_hbm.at[pl.ds(base, _SEG)])


_sc_kernel = functools.partial(
    pl.kernel,
    out_type=jax.ShapeDtypeStruct((_N,), jnp.float32),
    mesh=plsc.VectorSubcoreMesh(core_axis_name="c", subcore_axis_name="s",
                                num_cores=_NC, num_subcores=_NS),
    scratch_types=[
        pltpu.VMEM((_N,), jnp.float32),
        pltpu.VMEM((_L, _SEG), jnp.float32),
        pltpu.VMEM((_SEG,), jnp.float32),
        pltpu.VMEM((_SEG,), jnp.float32),
        pltpu.VMEM((_SEG,), jnp.float32),
        pltpu.VMEM((_SEG,), jnp.float32),
        pltpu.VMEM((_SEG,), jnp.float32),
        pltpu.VMEM((_SEG,), jnp.float32),
        pltpu.SemaphoreType.DMA,
    ],
)(_sc_body)


def kernel(x, activation, input_gain, threshold, freq_act, lateral_weights,
           spikes):
    del freq_act
    sf = spikes.reshape(_N).astype(jnp.float32)
    w2 = lateral_weights.reshape(_N * _NW, _SEG)
    out = _sc_kernel(sf, x.reshape(_N), activation.reshape(_N),
                     input_gain.reshape(_N), threshold.reshape(_N), w2)
    return out.reshape(_SHAPE).astype(jnp.bool_)


# speedup vs baseline: 11.3772x; 11.3772x over previous
import functools

import jax
import jax.numpy as jnp
from jax import lax
from jax.experimental import pallas as pl
from jax.experimental.pallas import tpu as pltpu
from jax.experimental.pallas import tpu_sc as plsc

_SHAPE = (64, 128)
_N = _SHAPE[0] * _SHAPE[1]
_NC, _NS, _L = 2, 16, 16
_NW = _NC * _NS
_SEG = _N // _NW
_CHUNKS = _SEG // _L
_NBLK = _N // _L

_BETA = 0.9


def _any_nonzero(v):
    s = v[0]
    for j in range(1, _L):
        s = s + v[j]
    return s != 0.0


def _sc_body(sf_hbm, x_hbm, act_hbm, gain_hbm, thr_hbm, w_hbm, out_hbm,
             sp_v, row_v, acc_v, x_v, a_v, g_v, t_v, o_v, sem):
    wid = lax.axis_index("s") * _NC + lax.axis_index("c")
    base = wid * _SEG

    pltpu.sync_copy(sf_hbm, sp_v)
    pltpu.sync_copy(x_hbm.at[pl.ds(base, _SEG)], x_v)
    pltpu.sync_copy(act_hbm.at[pl.ds(base, _SEG)], a_v)
    pltpu.sync_copy(gain_hbm.at[pl.ds(base, _SEG)], g_v)
    pltpu.sync_copy(thr_hbm.at[pl.ds(base, _SEG)], t_v)

    def or_body(b, acc):
        sf = sp_v[pl.ds(b * _L, _L)]
        return jnp.maximum(acc, sf)

    anyv = lax.fori_loop(0, _NBLK, or_body, jnp.zeros((_L,), jnp.float32))
    has_spikes = _any_nonzero(anyv)

    zeros = jnp.zeros((_L,), jnp.float32)
    for k in range(_CHUNKS):
        acc_v[pl.ds(k * _L, _L)] = zeros

    @pl.when(has_spikes)
    def _heavy():
        def chunk_body(c, carry):
            blk = c * _L
            sf = sp_v[pl.ds(blk, _L)]

            @pl.when(_any_nonzero(sf))
            def _():
                for j in range(_L):
                    @pl.when(sf[j] != 0.0)
                    def _():
                        pltpu.sync_copy(
                            w_hbm.at[blk + j, pl.ds(base, _SEG)], row_v)

                        def add_chunk(k, cc):
                            sl = pl.ds(k * _L, _L)
                            acc_v[sl] = acc_v[sl] + row_v[sl]
                            return cc
                        lax.fori_loop(0, _CHUNKS, add_chunk, 0)
            return carry

        lax.fori_loop(0, _NBLK, chunk_body, 0)

    for k in range(_CHUNKS):
        sl = pl.ds(k * _L, _L)
        gg = g_v[sl]
        ig = gg + (1.0 - gg) * 0.2
        act = _BETA * a_v[sl] + (x_v[sl] + acc_v[sl]) * ig + 0.05
        o_v[sl] = jnp.where(act > t_v[sl], 1.0, 0.0)
    pltpu.sync_copy(o_v, out_hbm.at[pl.ds(base, _SEG)])


_sc_kernel = functools.partial(
    pl.kernel,
    out_type=jax.ShapeDtypeStruct((_N,), jnp.float32),
    mesh=plsc.VectorSubcoreMesh(core_axis_name="c", subcore_axis_name="s",
                                num_cores=_NC, num_subcores=_NS),
    scratch_types=[
        pltpu.VMEM((_N,), jnp.float32),
        pltpu.VMEM((_SEG,), jnp.float32),
        pltpu.VMEM((_SEG,), jnp.float32),
        pltpu.VMEM((_SEG,), jnp.float32),
        pltpu.VMEM((_SEG,), jnp.float32),
        pltpu.VMEM((_SEG,), jnp.float32),
        pltpu.VMEM((_SEG,), jnp.float32),
        pltpu.VMEM((_SEG,), jnp.float32),
        pltpu.SemaphoreType.DMA,
    ],
)(_sc_body)


def kernel(x, activation, input_gain, threshold, freq_act, lateral_weights,
           spikes):
    del freq_act
    sf = spikes.reshape(_N).astype(jnp.float32)
    out = _sc_kernel(sf, x.reshape(_N), activation.reshape(_N),
                     input_gain.reshape(_N), threshold.reshape(_N),
                     lateral_weights)
    return out.reshape(_SHAPE).astype(jnp.bool_)


# speedup vs baseline: 14.0488x; 1.2348x over previous
import functools

import jax
import jax.numpy as jnp
from jax import lax
from jax.experimental import pallas as pl
from jax.experimental.pallas import tpu as pltpu
from jax.experimental.pallas import tpu_sc as plsc

_SHAPE = (64, 128)
_N = _SHAPE[0] * _SHAPE[1]
_NC, _NS, _L = 2, 16, 16
_NW = _NC * _NS
_SEG = _N // _NW
_CHUNKS = _SEG // _L
_NBLK = _N // _L
_UNROLL = 8

_BETA = 0.9


def _any_nonzero(v):
    s = v[0]
    for j in range(1, _L):
        s = s + v[j]
    return s != 0.0


def _sc_body(sf_hbm, x_hbm, act_hbm, gain_hbm, thr_hbm, w_hbm, out_hbm,
             sp_v, row_v, acc_v, x_v, a_v, g_v, t_v, o_v, sem, sem2):
    wid = lax.axis_index("s") * _NC + lax.axis_index("c")
    base = wid * _SEG

    pltpu.sync_copy(sf_hbm, sp_v.at[pl.ds(0, _N)])
    cx = pltpu.async_copy(x_hbm.at[pl.ds(base, _SEG)], x_v, sem2)
    ca = pltpu.async_copy(act_hbm.at[pl.ds(base, _SEG)], a_v, sem2)
    cg = pltpu.async_copy(gain_hbm.at[pl.ds(base, _SEG)], g_v, sem2)
    ct = pltpu.async_copy(thr_hbm.at[pl.ds(base, _SEG)], t_v, sem2)

    def or_body(b, acc):
        for u in range(_UNROLL):
            acc = jnp.maximum(acc, sp_v[pl.ds((b * _UNROLL + u) * _L, _L)])
        return acc

    anyv = lax.fori_loop(0, _NBLK // _UNROLL, or_body,
                         jnp.zeros((_L,), jnp.float32))
    has_spikes = _any_nonzero(anyv)

    def zero_body(k, c):
        acc_v[pl.ds(k * _L, _L)] = jnp.zeros((_L,), jnp.float32)
        return c

    lax.fori_loop(0, _CHUNKS, zero_body, 0)

    @pl.when(has_spikes)
    def _heavy():
        def row_body(r, carry):
            s = sp_v[pl.ds(r, _L)][0]

            @pl.when(s != 0.0)
            def _():
                pltpu.sync_copy(w_hbm.at[r, pl.ds(base, _SEG)], row_v)

                def add_chunk(k, cc):
                    sl = pl.ds(k * _L, _L)
                    acc_v[sl] = acc_v[sl] + row_v[sl]
                    return cc
                lax.fori_loop(0, _CHUNKS, add_chunk, 0)
            return carry

        lax.fori_loop(0, _N, row_body, 0)

    cx.wait()
    ca.wait()
    cg.wait()
    ct.wait()

    def ew_body(k, c):
        sl = pl.ds(k * _L, _L)
        gg = g_v[sl]
        ig = gg + (1.0 - gg) * 0.2
        act = _BETA * a_v[sl] + (x_v[sl] + acc_v[sl]) * ig + 0.05
        o_v[sl] = jnp.where(act > t_v[sl], 1.0, 0.0)
        return c

    lax.fori_loop(0, _CHUNKS, ew_body, 0)
    pltpu.sync_copy(o_v, out_hbm.at[pl.ds(base, _SEG)])


_sc_kernel = functools.partial(
    pl.kernel,
    out_type=jax.ShapeDtypeStruct((_N,), jnp.float32),
    mesh=plsc.VectorSubcoreMesh(core_axis_name="c", subcore_axis_name="s",
                                num_cores=_NC, num_subcores=_NS),
    scratch_types=[
        pltpu.VMEM((_N + _L,), jnp.float32),
        pltpu.VMEM((_SEG,), jnp.float32),
        pltpu.VMEM((_SEG,), jnp.float32),
        pltpu.VMEM((_SEG,), jnp.float32),
        pltpu.VMEM((_SEG,), jnp.float32),
        pltpu.VMEM((_SEG,), jnp.float32),
        pltpu.VMEM((_SEG,), jnp.float32),
        pltpu.VMEM((_SEG,), jnp.float32),
        pltpu.SemaphoreType.DMA,
        pltpu.SemaphoreType.DMA,
    ],
)(_sc_body)


def kernel(x, activation, input_gain, threshold, freq_act, lateral_weights,
           spikes):
    del freq_act
    sf = spikes.reshape(_N).astype(jnp.float32)
    out = _sc_kernel(sf, x.reshape(_N), activation.reshape(_N),
                     input_gain.reshape(_N), threshold.reshape(_N),
                     lateral_weights)
    return out.reshape(_SHAPE).astype(jnp.bool_)
